# Initial kernel scaffold; baseline (speedup 1.0000x reference)
import functools

import jax
import jax.numpy as jnp
from jax import lax
from jax.experimental import pallas as pl
from jax.experimental.pallas import tpu as pltpu
from jax.experimental.pallas import tpu_sc as plsc

N_CTX = 16
CTX_DIM = 512
SEQ = 77
N_CLS = 1024

_INFO = plsc.get_sparse_core_info()
_NC = _INFO.num_cores
_NS = _INFO.num_subcores
_NW = _NC * _NS
_CPW = N_CLS // _NW
_NGATHER = 64


def _sc_kernel(idx_hbm, table_hbm, ctx_hbm, out_hbm, idx_v, ctx_v, g_v, sem):
    wid = lax.axis_index("s") * _NC + lax.axis_index("c")
    base = wid * _CPW
    pltpu.sync_copy(idx_hbm.at[pl.ds(base, _CPW)], idx_v)
    pltpu.sync_copy(ctx_hbm, ctx_v)

    def body(i, carry):
        c = base + i
        pltpu.async_copy(table_hbm.at[idx_v.at[i]], g_v, sem).wait()
        pltpu.sync_copy(g_v.at[pl.ds(0, 1)], out_hbm.at[c, pl.ds(0, 1)])
        pltpu.sync_copy(ctx_v, out_hbm.at[c, pl.ds(1, N_CTX)])
        pltpu.sync_copy(g_v.at[pl.ds(1, SEQ - N_CTX - 1)],
                        out_hbm.at[c, pl.ds(1 + N_CTX, SEQ - N_CTX - 1)])
        return carry

    lax.fori_loop(0, _CPW, body, 0)


@jax.jit
def _run(idx, token_embedding, ctx):
    mesh = plsc.VectorSubcoreMesh(core_axis_name="c", subcore_axis_name="s")
    f = functools.partial(
        pl.kernel,
        mesh=mesh,
        out_type=jax.ShapeDtypeStruct((N_CLS, SEQ, CTX_DIM), jnp.float32),
        scratch_types=[
            pltpu.VMEM((_CPW, _NGATHER), jnp.int32),
            pltpu.VMEM((N_CTX, CTX_DIM), jnp.float32),
            pltpu.VMEM((_NGATHER, CTX_DIM), jnp.float32),
            pltpu.SemaphoreType.DMA,
        ],
    )(_sc_kernel)
    return f(idx, token_embedding, ctx)


def kernel(tokenized_prompts, token_embedding, ctx):
    gi = jnp.concatenate(
        [tokenized_prompts[:, :1], tokenized_prompts[:, 1 + N_CTX:]], axis=1)
    gi = jnp.pad(gi, ((0, 0), (0, _NGATHER - gi.shape[1])))
    prompts = _run(gi, token_embedding, ctx)
    return prompts, tokenized_prompts


# speedup vs baseline: 1.0237x; 1.0237x over previous
import functools

import jax
import jax.numpy as jnp
from jax import lax
from jax.experimental import pallas as pl
from jax.experimental.pallas import tpu as pltpu
from jax.experimental.pallas import tpu_sc as plsc

N_CTX = 16
CTX_DIM = 512
SEQ = 77
N_CLS = 1024

_INFO = plsc.get_sparse_core_info()
_NC = _INFO.num_cores
_NS = _INFO.num_subcores
_NW = _NC * _NS
_CPW = N_CLS // _NW
_NGATHER = 64


def _sc_kernel(idx_hbm, table_hbm, ctx_hbm, out_hbm, idx_v, ctx_v, g_v, sem):
    wid = lax.axis_index("s") * _NC + lax.axis_index("c")
    base = wid * _CPW
    pltpu.sync_copy(idx_hbm.at[pl.ds(base, _CPW)], idx_v)
    pltpu.sync_copy(ctx_hbm, ctx_v)

    def body(i, carry):
        c = base + i
        pltpu.async_copy(table_hbm.at[idx_v.at[i]], g_v, sem).wait()
        pltpu.sync_copy(g_v.at[pl.ds(0, 1)], out_hbm.at[c, pl.ds(0, 1)])
        pltpu.sync_copy(ctx_v, out_hbm.at[c, pl.ds(1, N_CTX)])
        pltpu.sync_copy(g_v.at[pl.ds(1, SEQ - N_CTX - 1)],
                        out_hbm.at[c, pl.ds(1 + N_CTX, SEQ - N_CTX - 1)])
        return carry

    lax.fori_loop(0, _CPW, body, 0)


@jax.jit
def _run(idx, token_embedding, ctx):
    mesh = plsc.VectorSubcoreMesh(core_axis_name="c", subcore_axis_name="s")
    f = functools.partial(
        pl.kernel,
        mesh=mesh,
        compiler_params=pltpu.CompilerParams(use_tc_tiling_on_sc=False),
        out_type=jax.ShapeDtypeStruct((N_CLS, SEQ, CTX_DIM), jnp.float32),
        scratch_types=[
            pltpu.VMEM((_CPW, _NGATHER), jnp.int32),
            pltpu.VMEM((N_CTX, CTX_DIM), jnp.float32),
            pltpu.VMEM((_NGATHER, CTX_DIM), jnp.float32),
            pltpu.SemaphoreType.DMA,
        ],
    )(_sc_kernel)
    return f(idx, token_embedding, ctx)


def kernel(tokenized_prompts, token_embedding, ctx):
    gi = jnp.concatenate(
        [tokenized_prompts[:, :1], tokenized_prompts[:, 1 + N_CTX:]], axis=1)
    gi = jnp.pad(gi, ((0, 0), (0, _NGATHER - gi.shape[1])))
    prompts = _run(gi, token_embedding, ctx)
    return prompts, tokenized_prompts


# speedup vs baseline: 1.5117x; 1.4767x over previous
import functools

import jax
import jax.numpy as jnp
from jax import lax
from jax.experimental import pallas as pl
from jax.experimental.pallas import tpu as pltpu
from jax.experimental.pallas import tpu_sc as plsc

N_CTX = 16
CTX_DIM = 512
SEQ = 77
N_CLS = 1024
N_SUF = SEQ - N_CTX - 1

_INFO = plsc.get_sparse_core_info()
_NC = _INFO.num_cores
_NS = _INFO.num_subcores
_NW = _NC * _NS
_CPW = N_CLS // _NW
_IW = 80


def _sc_kernel(gi_hbm, ctx_hbm, table_hbm, out_hbm,
               idx_v, blk0, blk1, g0, g1, w0, w1):
    wid = lax.axis_index("s") * _NC + lax.axis_index("c")
    base = wid * _CPW
    pltpu.sync_copy(gi_hbm.at[pl.ds(base * _IW, _CPW * _IW)], idx_v)
    pltpu.sync_copy(ctx_hbm, blk0.at[pl.ds(1, N_CTX)])
    pltpu.sync_copy(ctx_hbm, blk1.at[pl.ds(1, N_CTX)])

    def gathers(o, blk, sem):
        p = pltpu.async_copy(table_hbm.at[idx_v.at[pl.ds(o, 1)]],
                             blk.at[pl.ds(0, 1)], sem)
        s = pltpu.async_copy(table_hbm.at[idx_v.at[pl.ds(o + 8, N_SUF)]],
                             blk.at[pl.ds(1 + N_CTX, N_SUF)], sem)
        return p, s

    def body(j, carry):
        a = base + 2 * j
        oa = (2 * j) * _IW

        @pl.when(j > 0)
        def _():
            pltpu.make_async_copy(blk0, out_hbm.at[a - 2], w0).wait()

        pa, sa = gathers(oa, blk0, g0)

        @pl.when(j > 0)
        def _():
            pltpu.make_async_copy(blk1, out_hbm.at[a - 1], w1).wait()

        pb, sb = gathers(oa + _IW, blk1, g1)

        pa.wait()
        sa.wait()
        pltpu.async_copy(blk0, out_hbm.at[a], w0)
        pb.wait()
        sb.wait()
        pltpu.async_copy(blk1, out_hbm.at[a + 1], w1)
        return carry

    lax.fori_loop(0, _CPW // 2, body, 0)
    pltpu.make_async_copy(blk0, out_hbm.at[base + _CPW - 2], w0).wait()
    pltpu.make_async_copy(blk1, out_hbm.at[base + _CPW - 1], w1).wait()


@jax.jit
def _run(gi, ctx, token_embedding):
    mesh = plsc.VectorSubcoreMesh(core_axis_name="c", subcore_axis_name="s")
    f = functools.partial(
        pl.kernel,
        mesh=mesh,
        compiler_params=pltpu.CompilerParams(use_tc_tiling_on_sc=False),
        out_type=jax.ShapeDtypeStruct((N_CLS, SEQ, CTX_DIM), jnp.float32),
        scratch_types=[
            pltpu.VMEM((_CPW * _IW,), jnp.int32),
            pltpu.VMEM((SEQ, CTX_DIM), jnp.float32),
            pltpu.VMEM((SEQ, CTX_DIM), jnp.float32),
            pltpu.SemaphoreType.DMA,
            pltpu.SemaphoreType.DMA,
            pltpu.SemaphoreType.DMA,
            pltpu.SemaphoreType.DMA,
        ],
    )(_sc_kernel)
    return f(gi, ctx, token_embedding)


def kernel(tokenized_prompts, token_embedding, ctx):
    gi = jnp.zeros((N_CLS, _IW), jnp.int32)
    gi = gi.at[:, 0].set(tokenized_prompts[:, 0])
    gi = gi.at[:, 8:8 + N_SUF].set(tokenized_prompts[:, 1 + N_CTX:])
    gi = gi.reshape(-1)
    prompts = _run(gi, ctx, token_embedding)
    return prompts, tokenized_prompts
